# baseline (speedup 1.0000x reference)
import functools

import jax
import jax.numpy as jnp
from jax import lax
from jax.experimental import pallas as pl
from jax.experimental.pallas import tpu as pltpu
from jax.experimental.pallas import tpu_sc as plsc

_NC = 2
_NS = 16
_NW = _NC * _NS
_CH = 80
_NBUF = 2


def _prep(x, V, comp, loop_w, b):
    N, D = x.shape
    BLK = 1000
    nb = N // BLK

    def body(comp_ref, x_ref, V_ref, loop_ref, b_ref, out_ref):
        r = pl.program_id(1)
        rc = jnp.minimum(r, 3)
        c0 = comp_ref[rc, 0]
        c1 = comp_ref[rc, 1]
        Wm = c0 * V_ref[0] + c1 * V_ref[1]
        Wm = jnp.where(r < 4, Wm, loop_ref[...])
        y = jnp.dot(x_ref[...], Wm, preferred_element_type=jnp.float32)
        y = y + jnp.where(r == 4, 1.0, 0.0) * b_ref[...]
        y = jnp.where(r == 5, jnp.zeros_like(y), y)
        out_ref[...] = y[None]

    return pl.pallas_call(
        body,
        grid=(nb, 6),
        in_specs=[
            pl.BlockSpec(memory_space=pltpu.SMEM),
            pl.BlockSpec((BLK, D), lambda bb, rr: (bb, 0)),
            pl.BlockSpec((2, D, D), lambda bb, rr: (0, 0, 0)),
            pl.BlockSpec((D, D), lambda bb, rr: (0, 0)),
            pl.BlockSpec((1, D), lambda bb, rr: (0, 0)),
        ],
        out_specs=pl.BlockSpec((1, BLK, D), lambda bb, rr: (rr, bb, 0)),
        out_shape=jax.ShapeDtypeStruct((6, N, D), jnp.float32),
    )(comp, x, V, loop_w, b.reshape(1, D))


def _gather_idx(edge_types, src, N):
    E = src.shape[0]
    rows = E // 128
    et2 = edge_types.reshape(rows, 128)
    s2 = src.reshape(rows, 128)

    def body(a_ref, b_ref, o_ref):
        o_ref[...] = a_ref[...] * N + b_ref[...]

    return pl.pallas_call(
        body,
        out_shape=jax.ShapeDtypeStruct((rows, 128), jnp.int32),
    )(et2, s2)


def _combine(parts):
    _, N, D = parts.shape
    BLK = 1000
    nb = N // BLK

    def body(p_ref, o_ref):
        o_ref[...] = jnp.maximum(p_ref[0] + p_ref[1], 0.0)

    return pl.pallas_call(
        body,
        grid=(nb,),
        in_specs=[pl.BlockSpec((2, BLK, D), lambda bb: (0, bb, 0))],
        out_specs=pl.BlockSpec((BLK, D), lambda bb: (bb, 0)),
        out_shape=jax.ShapeDtypeStruct((N, D), jnp.float32),
    )(parts)


def _final(parts, pred_w, pred_b):
    _, N, D = parts.shape
    BLK = 1000
    nb = N // BLK

    def body(pb_ref, p_ref, pw_ref, o_ref):
        h = jnp.maximum(p_ref[0] + p_ref[1], 0.0)
        lg = jnp.sum(h * pw_ref[...], axis=1, keepdims=True) + pb_ref[0]
        o_ref[...] = jnp.broadcast_to(jax.nn.sigmoid(lg), (BLK, D))

    return pl.pallas_call(
        body,
        grid=(nb,),
        in_specs=[
            pl.BlockSpec(memory_space=pltpu.SMEM),
            pl.BlockSpec((2, BLK, D), lambda bb: (0, bb, 0)),
            pl.BlockSpec((1, D), lambda bb: (0, 0)),
        ],
        out_specs=pl.BlockSpec((BLK, D), lambda bb: (bb, 0)),
        out_shape=jax.ShapeDtypeStruct((N, D), jnp.float32),
    )(pred_b, parts, pred_w.reshape(1, D))


def _sc_layer(table_flat, idx_r, dst_r, N, D):
    NCH, CH = idx_r.shape[1], idx_r.shape[2]
    NBLK8 = N // 8
    base_blocks = NBLK8 // _NS
    extra = NBLK8 - base_blocks * _NS
    MAIN = base_blocks * 8

    mesh = plsc.VectorSubcoreMesh(
        core_axis_name="c", subcore_axis_name="s",
        num_cores=_NC, num_subcores=_NS)

    BCH = 16
    NBLKC = NCH // BCH

    @functools.partial(
        pl.kernel,
        out_type=jax.ShapeDtypeStruct((_NC, N, D), jnp.float32),
        mesh=mesh,
        scratch_types=[
            pltpu.VMEM((2 * BCH, CH), jnp.int32),
            pltpu.VMEM((2 * BCH, CH), jnp.int32),
            [pltpu.VMEM((CH, D), jnp.float32) for _ in range(_NBUF)],
            pltpu.VMEM_SHARED((N, D), jnp.float32),
            [pltpu.SemaphoreType.DMA for _ in range(2 * _NBUF + 2)],
        ],
    )
    def k(table_hbm, idx_hbm, dst_hbm, out_hbm, ib, db, rows, acc, sems):
        c = lax.axis_index("c")
        s = lax.axis_index("s")
        w = c * _NS + s
        row0 = 8 * (s * base_blocks + jnp.minimum(s, extra))
        tbase = (4 + c) * N
        pltpu.sync_copy(table_hbm.at[pl.ds(tbase + row0, MAIN), :],
                        acc.at[pl.ds(row0, MAIN), :])

        @pl.when(s < extra)
        def _():
            pltpu.sync_copy(table_hbm.at[pl.ds(tbase + row0 + MAIN, 8), :],
                            acc.at[pl.ds(row0 + MAIN, 8), :])

        def ib_copy(blk, slot):
            return pltpu.make_async_copy(
                idx_hbm.at[w, pl.ds(blk * BCH, BCH)],
                ib.at[pl.ds(slot * BCH, BCH)], sems[2 + slot])

        def db_copy(blk, slot):
            return pltpu.make_async_copy(
                dst_hbm.at[w, pl.ds(blk * BCH, BCH)],
                db.at[pl.ds(slot * BCH, BCH)], sems[4 + slot])

        for t in range(min(2, NBLKC)):
            ib_copy(t, t).start()
            db_copy(t, t).start()
        plsc.subcore_barrier()

        ib_copy(0, 0).wait()
        db_copy(0, 0).wait()

        for kblk in range(NBLKC):
            slot = kblk % 2
            nslot = 1 - slot
            if 1 <= kblk and kblk + 1 < NBLKC:
                ib_copy(kblk + 1, nslot).start()
                db_copy(kblk + 1, nslot).start()

            def body(g, carry, kblk=kblk, slot=slot, nslot=nslot):
                for b in range(_NBUF):
                    r = g * _NBUF + b
                    pltpu.async_copy(
                        table_hbm.at[ib.at[slot * BCH + r]],
                        rows[b], sems[b]).wait()
                    pltpu.sync_copy(rows[b], acc.at[db.at[slot * BCH + r]],
                                    add=True)
                if kblk + 1 < NBLKC:
                    @pl.when(g == BCH // _NBUF - 1)
                    def _():
                        ib_copy(kblk + 1, nslot).wait()
                        db_copy(kblk + 1, nslot).wait()
                return carry

            lax.fori_loop(0, BCH // _NBUF, body, 0)
        plsc.subcore_barrier()
        pltpu.sync_copy(acc.at[pl.ds(row0, MAIN), :],
                        out_hbm.at[c, pl.ds(row0, MAIN), :])

        @pl.when(s < extra)
        def _():
            pltpu.sync_copy(acc.at[pl.ds(row0 + MAIN, 8), :],
                            out_hbm.at[c, pl.ds(row0 + MAIN, 8), :])

    return k(table_flat, idx_r, dst_r)


def kernel(features, edge_index, edge_types, V1, comp1, loop_w1, b1,
           V2, comp2, loop_w2, b2, pred_w, pred_b):
    N, D = features.shape
    E = edge_index.shape[1]
    src = edge_index[0]
    dst = edge_index[1]

    epw = E // _NW
    nch = (epw + _CH - 1) // _CH
    nch = (nch + 15) // 16 * 16
    pad = nch * _CH - epw
    idx = _gather_idx(edge_types, src, N).reshape(_NW, epw)
    idx_r = jnp.pad(idx, ((0, 0), (0, pad)),
                    constant_values=5 * N).reshape(_NW, nch, _CH)
    dst_r = jnp.pad(dst.reshape(_NW, epw),
                    ((0, 0), (0, pad))).reshape(_NW, nch, _CH)

    table1 = _prep(features, V1, comp1, loop_w1, b1).reshape(6 * N, D)
    parts1 = _sc_layer(table1, idx_r, dst_r, N, D)
    h1 = _combine(parts1)

    table2 = _prep(h1, V2, comp2, loop_w2, b2).reshape(6 * N, D)
    parts2 = _sc_layer(table2, idx_r, dst_r, N, D)

    out = _final(parts2, pred_w, pred_b)
    return out[:, 0]


# speedup vs baseline: 2.9408x; 2.9408x over previous
import functools

import jax
import jax.numpy as jnp
from jax import lax
from jax.experimental import pallas as pl
from jax.experimental.pallas import tpu as pltpu
from jax.experimental.pallas import tpu_sc as plsc

_NC = 2
_NS = 16
_NW = _NC * _NS
_CH = 80
_NBUF = 2


def _prep(x, V, comp, loop_w, b):
    N, D = x.shape
    BLK = 1000
    nb = N // BLK

    def body(comp_ref, x_ref, V_ref, loop_ref, b_ref, out_ref):
        r = pl.program_id(1)
        rc = jnp.minimum(r, 3)
        c0 = comp_ref[rc, 0]
        c1 = comp_ref[rc, 1]
        Wm = c0 * V_ref[0] + c1 * V_ref[1]
        Wm = jnp.where(r < 4, Wm, loop_ref[...])
        y = jnp.dot(x_ref[...], Wm, preferred_element_type=jnp.float32)
        y = y + jnp.where(r == 4, 1.0, 0.0) * b_ref[...]
        y = jnp.where(r == 5, jnp.zeros_like(y), y)
        out_ref[...] = y[None]

    return pl.pallas_call(
        body,
        grid=(nb, 6),
        in_specs=[
            pl.BlockSpec(memory_space=pltpu.SMEM),
            pl.BlockSpec((BLK, D), lambda bb, rr: (bb, 0)),
            pl.BlockSpec((2, D, D), lambda bb, rr: (0, 0, 0)),
            pl.BlockSpec((D, D), lambda bb, rr: (0, 0)),
            pl.BlockSpec((1, D), lambda bb, rr: (0, 0)),
        ],
        out_specs=pl.BlockSpec((1, BLK, D), lambda bb, rr: (rr, bb, 0)),
        out_shape=jax.ShapeDtypeStruct((6, N, D), jnp.float32),
    )(comp, x, V, loop_w, b.reshape(1, D))


def _gather_idx(edge_types, src, dst, N):
    E = src.shape[0]
    rows = E // 128
    et2 = edge_types.reshape(rows, 128)
    s2 = src.reshape(rows, 128)
    d2 = dst.reshape(rows, 128)

    def body(a_ref, b_ref, c_ref, o_ref):
        o_ref[...] = (a_ref[...] * N + b_ref[...]) | (c_ref[...] << 16)

    return pl.pallas_call(
        body,
        out_shape=jax.ShapeDtypeStruct((rows, 128), jnp.int32),
    )(et2, s2, d2)


def _combine(parts):
    _, N, D = parts.shape
    BLK = 1000
    nb = N // BLK

    def body(p_ref, o_ref):
        o_ref[...] = jnp.maximum(p_ref[0] + p_ref[1], 0.0)

    return pl.pallas_call(
        body,
        grid=(nb,),
        in_specs=[pl.BlockSpec((2, BLK, D), lambda bb: (0, bb, 0))],
        out_specs=pl.BlockSpec((BLK, D), lambda bb: (bb, 0)),
        out_shape=jax.ShapeDtypeStruct((N, D), jnp.float32),
    )(parts)


def _final(parts, pred_w, pred_b):
    _, N, D = parts.shape
    BLK = 1000
    nb = N // BLK

    def body(pb_ref, p_ref, pw_ref, o_ref):
        h = jnp.maximum(p_ref[0] + p_ref[1], 0.0)
        lg = jnp.sum(h * pw_ref[...], axis=1, keepdims=True) + pb_ref[0]
        o_ref[...] = jnp.broadcast_to(jax.nn.sigmoid(lg), (BLK, D))

    return pl.pallas_call(
        body,
        grid=(nb,),
        in_specs=[
            pl.BlockSpec(memory_space=pltpu.SMEM),
            pl.BlockSpec((2, BLK, D), lambda bb: (0, bb, 0)),
            pl.BlockSpec((1, D), lambda bb: (0, 0)),
        ],
        out_specs=pl.BlockSpec((BLK, D), lambda bb: (bb, 0)),
        out_shape=jax.ShapeDtypeStruct((N, D), jnp.float32),
    )(pred_b, parts, pred_w.reshape(1, D))


def _sc_layer(table_flat, iv_r, N, D):
    NCH, CH = iv_r.shape[1], iv_r.shape[2]
    NBLK8 = N // 8
    base_blocks = NBLK8 // _NS
    extra = NBLK8 - base_blocks * _NS
    MAIN = base_blocks * 8

    mesh = plsc.VectorSubcoreMesh(
        core_axis_name="c", subcore_axis_name="s",
        num_cores=_NC, num_subcores=_NS)

    NV = CH // 16

    @functools.partial(
        pl.kernel,
        out_type=jax.ShapeDtypeStruct((_NC, N, D), jnp.float32),
        mesh=mesh,
        scratch_types=[
            pltpu.VMEM((NCH, CH), jnp.int32),
            [pltpu.VMEM((1, CH), jnp.int32) for _ in range(_NBUF)],
            pltpu.VMEM((1, CH), jnp.int32),
            [pltpu.VMEM((CH, D), jnp.float32) for _ in range(_NBUF)],
            pltpu.VMEM_SHARED((N, D), jnp.float32),
            [pltpu.SemaphoreType.DMA for _ in range(_NBUF)],
        ],
    )
    def k(table_hbm, iv_hbm, out_hbm, iv, gx, dx, rows, acc, sems):
        c = lax.axis_index("c")
        s = lax.axis_index("s")
        w = c * _NS + s
        row0 = 8 * (s * base_blocks + jnp.minimum(s, extra))
        tbase = (4 + c) * N
        pltpu.sync_copy(table_hbm.at[pl.ds(tbase + row0, MAIN), :],
                        acc.at[pl.ds(row0, MAIN), :])

        @pl.when(s < extra)
        def _():
            pltpu.sync_copy(table_hbm.at[pl.ds(tbase + row0 + MAIN, 8), :],
                            acc.at[pl.ds(row0 + MAIN, 8), :])

        pltpu.sync_copy(iv_hbm.at[w], iv)
        plsc.subcore_barrier()

        def unpack_lo(i, dref):
            for j in range(NV):
                sl = pl.ds(j * 16, 16)
                dref[0, sl] = iv[i, sl] & 0xFFFF

        def unpack_hi(i, dref):
            for j in range(NV):
                sl = pl.ds(j * 16, 16)
                dref[0, sl] = iv[i, sl] >> 16

        def gather(b):
            return pltpu.make_async_copy(
                table_hbm.at[gx[b].at[0]], rows[b], sems[b])

        for b in range(_NBUF):
            unpack_lo(b, gx[b])
            gather(b).start()

        def body(g, carry):
            for b in range(_NBUF):
                i = g * _NBUF + b

                @pl.when(i < NCH)
                def _():
                    gather(b).wait()
                    unpack_hi(i, dx)
                    pltpu.sync_copy(rows[b], acc.at[dx.at[0]], add=True)

                    @pl.when(i + _NBUF < NCH)
                    def _():
                        unpack_lo(i + _NBUF, gx[b])
                        gather(b).start()
            return carry

        lax.fori_loop(0, (NCH + _NBUF - 1) // _NBUF, body, 0)
        plsc.subcore_barrier()
        pltpu.sync_copy(acc.at[pl.ds(row0, MAIN), :],
                        out_hbm.at[c, pl.ds(row0, MAIN), :])

        @pl.when(s < extra)
        def _():
            pltpu.sync_copy(acc.at[pl.ds(row0 + MAIN, 8), :],
                            out_hbm.at[c, pl.ds(row0 + MAIN, 8), :])

    return k(table_flat, iv_r)


def kernel(features, edge_index, edge_types, V1, comp1, loop_w1, b1,
           V2, comp2, loop_w2, b2, pred_w, pred_b):
    N, D = features.shape
    E = edge_index.shape[1]
    src = edge_index[0]
    dst = edge_index[1]

    epw = E // _NW
    nch = (epw + _CH - 1) // _CH
    pad = nch * _CH - epw
    iv = _gather_idx(edge_types, src, dst, N).reshape(_NW, epw)
    iv_r = jnp.pad(iv, ((0, 0), (0, pad)),
                   constant_values=5 * N).reshape(_NW, nch, _CH)

    table1 = _prep(features, V1, comp1, loop_w1, b1).reshape(6 * N, D)
    parts1 = _sc_layer(table1, iv_r, N, D)
    h1 = _combine(parts1)

    table2 = _prep(h1, V2, comp2, loop_w2, b2).reshape(6 * N, D)
    parts2 = _sc_layer(table2, iv_r, N, D)

    out = _final(parts2, pred_w, pred_b)
    return out[:, 0]


# speedup vs baseline: 3.2720x; 1.1126x over previous
import functools

import jax
import jax.numpy as jnp
from jax import lax
from jax.experimental import pallas as pl
from jax.experimental.pallas import tpu as pltpu
from jax.experimental.pallas import tpu_sc as plsc

_NC = 2
_NS = 16
_NW = _NC * _NS
_CH = 80
_NBUF = 3


def _prep(x, V, comp, loop_w, b):
    N, D = x.shape
    BLK = 1000
    nb = N // BLK

    def body(comp_ref, x_ref, V_ref, loop_ref, b_ref, out_ref):
        r = pl.program_id(1)
        rc = jnp.minimum(r, 3)
        c0 = comp_ref[rc, 0]
        c1 = comp_ref[rc, 1]
        Wm = c0 * V_ref[0] + c1 * V_ref[1]
        Wm = jnp.where(r < 4, Wm, loop_ref[...])
        y = jnp.dot(x_ref[...], Wm, preferred_element_type=jnp.float32)
        y = y + jnp.where(r == 4, 1.0, 0.0) * b_ref[...]
        y = jnp.where(r == 5, jnp.zeros_like(y), y)
        out_ref[...] = y[None]

    return pl.pallas_call(
        body,
        grid=(nb, 6),
        in_specs=[
            pl.BlockSpec(memory_space=pltpu.SMEM),
            pl.BlockSpec((BLK, D), lambda bb, rr: (bb, 0)),
            pl.BlockSpec((2, D, D), lambda bb, rr: (0, 0, 0)),
            pl.BlockSpec((D, D), lambda bb, rr: (0, 0)),
            pl.BlockSpec((1, D), lambda bb, rr: (0, 0)),
        ],
        out_specs=pl.BlockSpec((1, BLK, D), lambda bb, rr: (rr, bb, 0)),
        out_shape=jax.ShapeDtypeStruct((6, N, D), jnp.float32),
    )(comp, x, V, loop_w, b.reshape(1, D))


def _gather_idx(edge_types, src, dst, N):
    E = src.shape[0]
    rows = E // 128
    et2 = edge_types.reshape(rows, 128)
    s2 = src.reshape(rows, 128)
    d2 = dst.reshape(rows, 128)

    def body(a_ref, b_ref, c_ref, o_ref):
        o_ref[...] = (a_ref[...] * N + b_ref[...]) | (c_ref[...] << 16)

    return pl.pallas_call(
        body,
        out_shape=jax.ShapeDtypeStruct((rows, 128), jnp.int32),
    )(et2, s2, d2)


def _combine(parts):
    _, N, D = parts.shape
    BLK = 1000
    nb = N // BLK

    def body(p_ref, o_ref):
        o_ref[...] = jnp.maximum(p_ref[0] + p_ref[1], 0.0)

    return pl.pallas_call(
        body,
        grid=(nb,),
        in_specs=[pl.BlockSpec((2, BLK, D), lambda bb: (0, bb, 0))],
        out_specs=pl.BlockSpec((BLK, D), lambda bb: (bb, 0)),
        out_shape=jax.ShapeDtypeStruct((N, D), jnp.float32),
    )(parts)


def _final(parts, pred_w, pred_b):
    _, N, D = parts.shape
    BLK = 1000
    nb = N // BLK

    def body(pb_ref, p_ref, pw_ref, o_ref):
        h = jnp.maximum(p_ref[0] + p_ref[1], 0.0)
        lg = jnp.sum(h * pw_ref[...], axis=1, keepdims=True) + pb_ref[0]
        o_ref[...] = jnp.broadcast_to(jax.nn.sigmoid(lg), (BLK, D))

    return pl.pallas_call(
        body,
        grid=(nb,),
        in_specs=[
            pl.BlockSpec(memory_space=pltpu.SMEM),
            pl.BlockSpec((2, BLK, D), lambda bb: (0, bb, 0)),
            pl.BlockSpec((1, D), lambda bb: (0, 0)),
        ],
        out_specs=pl.BlockSpec((BLK, D), lambda bb: (bb, 0)),
        out_shape=jax.ShapeDtypeStruct((N, D), jnp.float32),
    )(pred_b, parts, pred_w.reshape(1, D))


def _sc_layer(table_flat, iv_r, N, D):
    CH = _CH
    NCH = iv_r.shape[1] // CH
    NBLK8 = N // 8
    base_blocks = NBLK8 // _NS
    extra = NBLK8 - base_blocks * _NS
    MAIN = base_blocks * 8

    mesh = plsc.VectorSubcoreMesh(
        core_axis_name="c", subcore_axis_name="s",
        num_cores=_NC, num_subcores=_NS)

    NV = CH // 16

    @functools.partial(
        pl.kernel,
        out_type=jax.ShapeDtypeStruct((_NC, N, D), jnp.float32),
        mesh=mesh,
        scratch_types=[
            pltpu.VMEM((NCH * CH,), jnp.int32),
            [pltpu.VMEM((1, CH), jnp.int32) for _ in range(_NBUF)],
            [pltpu.VMEM((1, CH), jnp.int32) for _ in range(_NBUF)],
            [pltpu.VMEM((CH, D), jnp.float32) for _ in range(_NBUF)],
            pltpu.VMEM_SHARED((N, D), jnp.float32),
            [pltpu.SemaphoreType.DMA for _ in range(2 * _NBUF)],
        ],
    )
    def k(table_hbm, iv_hbm, out_hbm, iv, gx, dx, rows, acc, sems):
        c = lax.axis_index("c")
        s = lax.axis_index("s")
        w = c * _NS + s
        row0 = 8 * (s * base_blocks + jnp.minimum(s, extra))
        tbase = (4 + c) * N
        pltpu.sync_copy(table_hbm.at[pl.ds(tbase + row0, MAIN), :],
                        acc.at[pl.ds(row0, MAIN), :])

        @pl.when(s < extra)
        def _():
            pltpu.sync_copy(table_hbm.at[pl.ds(tbase + row0 + MAIN, 8), :],
                            acc.at[pl.ds(row0 + MAIN, 8), :])

        pltpu.sync_copy(iv_hbm.at[w], iv)
        plsc.subcore_barrier()

        def unpack_lo(i, dref):
            for j in range(NV):
                dref[0, pl.ds(j * 16, 16)] = iv[pl.ds(i * CH + j * 16, 16)] & 0xFFFF

        def unpack_hi(i, dref):
            for j in range(NV):
                dref[0, pl.ds(j * 16, 16)] = iv[pl.ds(i * CH + j * 16, 16)] >> 16

        def gather(b):
            return pltpu.make_async_copy(
                table_hbm.at[gx[b].at[0]], rows[b], sems[b])

        def scatter(b):
            return pltpu.make_async_copy(
                rows[b], acc.at[dx[b].at[0]], sems[_NBUF + b])

        for b in range(_NBUF - 1):
            unpack_lo(b, gx[b])
            gather(b).start()

        def body(g, carry):
            for b in range(_NBUF):
                k = g * _NBUF + b
                b2 = (b + 2) % _NBUF

                @pl.when(k < NCH)
                def _():
                    gather(b).wait()
                    unpack_hi(k, dx[b])
                    pltpu.async_copy(rows[b], acc.at[dx[b].at[0]],
                                     sems[_NBUF + b], add=True)

                    @pl.when(k >= 1)
                    def _():
                        scatter(b2).wait()

                    @pl.when(k + 2 < NCH)
                    def _():
                        unpack_lo(k + 2, gx[b2])
                        gather(b2).start()
            return carry

        lax.fori_loop(0, (NCH + _NBUF - 1) // _NBUF, body, 0)
        scatter((NCH - 1) % _NBUF).wait()
        plsc.subcore_barrier()
        pltpu.sync_copy(acc.at[pl.ds(row0, MAIN), :],
                        out_hbm.at[c, pl.ds(row0, MAIN), :])

        @pl.when(s < extra)
        def _():
            pltpu.sync_copy(acc.at[pl.ds(row0 + MAIN, 8), :],
                            out_hbm.at[c, pl.ds(row0 + MAIN, 8), :])

    return k(table_flat, iv_r)


def kernel(features, edge_index, edge_types, V1, comp1, loop_w1, b1,
           V2, comp2, loop_w2, b2, pred_w, pred_b):
    N, D = features.shape
    E = edge_index.shape[1]
    src = edge_index[0]
    dst = edge_index[1]

    epw = E // _NW
    nch = (epw + _CH - 1) // _CH
    pad = nch * _CH - epw
    iv = _gather_idx(edge_types, src, dst, N).reshape(_NW, epw)
    iv_r = jnp.pad(iv, ((0, 0), (0, pad)), constant_values=5 * N)

    table1 = _prep(features, V1, comp1, loop_w1, b1).reshape(6 * N, D)
    parts1 = _sc_layer(table1, iv_r, N, D)
    h1 = _combine(parts1)

    table2 = _prep(h1, V2, comp2, loop_w2, b2).reshape(6 * N, D)
    parts2 = _sc_layer(table2, iv_r, N, D)

    out = _final(parts2, pred_w, pred_b)
    return out[:, 0]


# speedup vs baseline: 3.3386x; 1.0203x over previous
import functools

import jax
import jax.numpy as jnp
from jax import lax
from jax.experimental import pallas as pl
from jax.experimental.pallas import tpu as pltpu
from jax.experimental.pallas import tpu_sc as plsc

_NC = 2
_NS = 16
_NW = _NC * _NS
_CH = 80
_NBUF = 3


def _prep(x, V, comp, loop_w, b):
    N, D = x.shape
    BLK = 1000
    nb = N // BLK

    def body(comp_ref, x_ref, V_ref, loop_ref, b_ref, out_ref):
        r = pl.program_id(1)
        rc = jnp.minimum(r, 3)
        c0 = comp_ref[rc, 0]
        c1 = comp_ref[rc, 1]
        Wm = c0 * V_ref[0] + c1 * V_ref[1]
        Wm = jnp.where(r < 4, Wm, loop_ref[...])
        y = jnp.dot(x_ref[...], Wm, preferred_element_type=jnp.float32)
        y = y + jnp.where(r == 4, 1.0, 0.0) * b_ref[...]
        y = jnp.where(r == 5, jnp.zeros_like(y), y)
        out_ref[...] = y[None]

    return pl.pallas_call(
        body,
        grid=(nb, 6),
        in_specs=[
            pl.BlockSpec(memory_space=pltpu.SMEM),
            pl.BlockSpec((BLK, D), lambda bb, rr: (bb, 0)),
            pl.BlockSpec((2, D, D), lambda bb, rr: (0, 0, 0)),
            pl.BlockSpec((D, D), lambda bb, rr: (0, 0)),
            pl.BlockSpec((1, D), lambda bb, rr: (0, 0)),
        ],
        out_specs=pl.BlockSpec((1, BLK, D), lambda bb, rr: (rr, bb, 0)),
        out_shape=jax.ShapeDtypeStruct((6, N, D), jnp.float32),
    )(comp, x, V, loop_w, b.reshape(1, D))


def _gather_idx(edge_types, src, dst, N):
    E = src.shape[0]
    rows = E // 128
    et2 = edge_types.reshape(rows, 128)
    s2 = src.reshape(rows, 128)
    d2 = dst.reshape(rows, 128)

    def body(a_ref, b_ref, c_ref, o_ref):
        o_ref[...] = (a_ref[...] * N + b_ref[...]) | (c_ref[...] << 16)

    return pl.pallas_call(
        body,
        out_shape=jax.ShapeDtypeStruct((rows, 128), jnp.int32),
    )(et2, s2, d2)


def _prep_from_parts(parts, V, comp, loop_w, b):
    _, N, D = parts.shape
    BLK = 1000
    nb = N // BLK

    def body(comp_ref, p_ref, V_ref, loop_ref, b_ref, out_ref):
        r = pl.program_id(1)
        x = jnp.maximum(p_ref[0] + p_ref[1], 0.0)
        rc = jnp.minimum(r, 3)
        c0 = comp_ref[rc, 0]
        c1 = comp_ref[rc, 1]
        Wm = c0 * V_ref[0] + c1 * V_ref[1]
        Wm = jnp.where(r < 4, Wm, loop_ref[...])
        y = jnp.dot(x, Wm, preferred_element_type=jnp.float32)
        y = y + jnp.where(r == 4, 1.0, 0.0) * b_ref[...]
        y = jnp.where(r == 5, jnp.zeros_like(y), y)
        out_ref[...] = y[None]

    return pl.pallas_call(
        body,
        grid=(nb, 6),
        in_specs=[
            pl.BlockSpec(memory_space=pltpu.SMEM),
            pl.BlockSpec((2, BLK, D), lambda bb, rr: (0, bb, 0)),
            pl.BlockSpec((2, D, D), lambda bb, rr: (0, 0, 0)),
            pl.BlockSpec((D, D), lambda bb, rr: (0, 0)),
            pl.BlockSpec((1, D), lambda bb, rr: (0, 0)),
        ],
        out_specs=pl.BlockSpec((1, BLK, D), lambda bb, rr: (rr, bb, 0)),
        out_shape=jax.ShapeDtypeStruct((6, N, D), jnp.float32),
    )(comp, parts, V, loop_w, b.reshape(1, D))


def _final(parts, pred_w, pred_b):
    _, N, D = parts.shape
    BLK = 1000
    nb = N // BLK

    def body(pb_ref, p_ref, pw_ref, o_ref):
        h = jnp.maximum(p_ref[0] + p_ref[1], 0.0)
        lg = jnp.sum(h * pw_ref[...], axis=1, keepdims=True) + pb_ref[0]
        o_ref[...] = jnp.broadcast_to(jax.nn.sigmoid(lg), (BLK, D))

    return pl.pallas_call(
        body,
        grid=(nb,),
        in_specs=[
            pl.BlockSpec(memory_space=pltpu.SMEM),
            pl.BlockSpec((2, BLK, D), lambda bb: (0, bb, 0)),
            pl.BlockSpec((1, D), lambda bb: (0, 0)),
        ],
        out_specs=pl.BlockSpec((BLK, D), lambda bb: (bb, 0)),
        out_shape=jax.ShapeDtypeStruct((N, D), jnp.float32),
    )(pred_b, parts, pred_w.reshape(1, D))


def _sc_layer(table_flat, iv_r, N, D):
    CH = _CH
    NCH = iv_r.shape[1] // CH
    NBLK8 = N // 8
    base_blocks = NBLK8 // _NS
    extra = NBLK8 - base_blocks * _NS
    MAIN = base_blocks * 8

    mesh = plsc.VectorSubcoreMesh(
        core_axis_name="c", subcore_axis_name="s",
        num_cores=_NC, num_subcores=_NS)

    NV = CH // 16

    @functools.partial(
        pl.kernel,
        out_type=jax.ShapeDtypeStruct((_NC, N, D), jnp.float32),
        mesh=mesh,
        scratch_types=[
            pltpu.VMEM((NCH * CH,), jnp.int32),
            [pltpu.VMEM((1, CH), jnp.int32) for _ in range(_NBUF)],
            [pltpu.VMEM((1, CH), jnp.int32) for _ in range(_NBUF)],
            [pltpu.VMEM((CH, D), jnp.float32) for _ in range(_NBUF)],
            pltpu.VMEM_SHARED((N, D), jnp.float32),
            [pltpu.SemaphoreType.DMA for _ in range(2 * _NBUF)],
        ],
    )
    def k(table_hbm, iv_hbm, out_hbm, iv, gx, dx, rows, acc, sems):
        c = lax.axis_index("c")
        s = lax.axis_index("s")
        w = c * _NS + s
        row0 = 8 * (s * base_blocks + jnp.minimum(s, extra))
        tbase = (4 + c) * N
        pltpu.sync_copy(table_hbm.at[pl.ds(tbase + row0, MAIN), :],
                        acc.at[pl.ds(row0, MAIN), :])

        @pl.when(s < extra)
        def _():
            pltpu.sync_copy(table_hbm.at[pl.ds(tbase + row0 + MAIN, 8), :],
                            acc.at[pl.ds(row0 + MAIN, 8), :])

        pltpu.sync_copy(iv_hbm.at[w], iv)
        plsc.subcore_barrier()

        def unpack_lo(i, dref):
            for j in range(NV):
                dref[0, pl.ds(j * 16, 16)] = iv[pl.ds(i * CH + j * 16, 16)] & 0xFFFF

        def unpack_hi(i, dref):
            for j in range(NV):
                dref[0, pl.ds(j * 16, 16)] = iv[pl.ds(i * CH + j * 16, 16)] >> 16

        def gather(b):
            return pltpu.make_async_copy(
                table_hbm.at[gx[b].at[0]], rows[b], sems[b])

        def scatter(b):
            return pltpu.make_async_copy(
                rows[b], acc.at[dx[b].at[0]], sems[_NBUF + b])

        for b in range(_NBUF - 1):
            unpack_lo(b, gx[b])
            gather(b).start()

        def body(g, carry):
            for b in range(_NBUF):
                k = g * _NBUF + b
                b2 = (b + 2) % _NBUF

                @pl.when(k < NCH)
                def _():
                    gather(b).wait()
                    unpack_hi(k, dx[b])
                    pltpu.async_copy(rows[b], acc.at[dx[b].at[0]],
                                     sems[_NBUF + b], add=True)

                    @pl.when(k >= 1)
                    def _():
                        scatter(b2).wait()

                    @pl.when(k + 2 < NCH)
                    def _():
                        unpack_lo(k + 2, gx[b2])
                        gather(b2).start()
            return carry

        lax.fori_loop(0, (NCH + _NBUF - 1) // _NBUF, body, 0)
        scatter((NCH - 1) % _NBUF).wait()
        plsc.subcore_barrier()
        pltpu.sync_copy(acc.at[pl.ds(row0, MAIN), :],
                        out_hbm.at[c, pl.ds(row0, MAIN), :])

        @pl.when(s < extra)
        def _():
            pltpu.sync_copy(acc.at[pl.ds(row0 + MAIN, 8), :],
                            out_hbm.at[c, pl.ds(row0 + MAIN, 8), :])

    return k(table_flat, iv_r)


def kernel(features, edge_index, edge_types, V1, comp1, loop_w1, b1,
           V2, comp2, loop_w2, b2, pred_w, pred_b):
    N, D = features.shape
    E = edge_index.shape[1]
    src = edge_index[0]
    dst = edge_index[1]

    epw = E // _NW
    nch = (epw + _CH - 1) // _CH
    pad = nch * _CH - epw
    iv = _gather_idx(edge_types, src, dst, N).reshape(_NW, epw)
    iv_r = jnp.pad(iv, ((0, 0), (0, pad)), constant_values=5 * N)

    table1 = _prep(features, V1, comp1, loop_w1, b1).reshape(6 * N, D)
    parts1 = _sc_layer(table1, iv_r, N, D)

    table2 = _prep_from_parts(parts1, V2, comp2, loop_w2, b2).reshape(6 * N, D)
    parts2 = _sc_layer(table2, iv_r, N, D)

    out = _final(parts2, pred_w, pred_b)
    return out[:, 0]
